# Initial kernel scaffold; baseline (speedup 1.0000x reference)
import jax
import jax.numpy as jnp
from jax.experimental import pallas as pl


def kernel(class_logits, box_regression, proposals):
    raise NotImplementedError("write your pallas kernel here")


# speedup vs baseline: 11.4698x; 11.4698x over previous
import functools

import jax
import jax.numpy as jnp
from jax import lax
from jax.experimental import pallas as pl
from jax.experimental.pallas import tpu as pltpu
from jax.experimental.pallas import tpu_sc as plsc

N = 20000
C = 91
KW = 96
SCORE_THRESH = 0.05
NMS_THRESH = 0.5
DET_PER_IMG = 100
PRE_NMS = 4096
IMG_H = 800.0
IMG_W = 1333.0
OFF = IMG_W + 2.0
BBOX_XFORM_CLIP = 4.135166556742356

ROWS_BLK = 1000
NEG_INF = float("-inf")


def _softmax_body(x_ref, out_ref):
    x = x_ref[...]
    m = jnp.max(x, axis=1, keepdims=True)
    e = jnp.exp(x - m)
    s = jnp.sum(e, axis=1, keepdims=True)
    p = e / s
    col = lax.broadcasted_iota(jnp.int32, x.shape, 1)
    ok = (p > SCORE_THRESH) & (col >= 1) & (col <= C - 1)
    key = lax.bitcast_convert_type(p, jnp.int32)
    out_ref[...] = jnp.where(ok, key, 0)


def _score_keys(class_logits):
    xpad = jnp.pad(class_logits, ((0, 0), (0, KW - C)), constant_values=-1e30)
    return pl.pallas_call(
        _softmax_body,
        grid=(N // ROWS_BLK,),
        in_specs=[pl.BlockSpec((ROWS_BLK, KW), lambda i: (i, 0))],
        out_specs=pl.BlockSpec((ROWS_BLK, KW), lambda i: (i, 0)),
        out_shape=jax.ShapeDtypeStruct((N, KW), jnp.int32),
    )(xpad)


NW = 32
WPT = (N * KW) // NW
VPT = WPT // 16
CCAP = 8192
SELCAP = 288
TOPT = 256
KBASE = 0x3D4C0000
NCAND = NW * SELCAP


def _sc_body(keys_hbm, breg_hbm, prop_hbm,
             ox1_hbm, oy1_hbm, ox2_hbm, oy2_hbm, osc_hbm, olb_hbm,
             keys_v, ckey, cpos, hist, selpos, selkey,
             gidx0, gidx1, gidx2, pidx0, pidx1, pidx2,
             rel, prp, ob0, ob1, ob2, ob3, ob4, ob5, sem):
    gidx = (gidx0, gidx1, gidx2)
    pidx = (pidx0, pidx1, pidx2)
    ob = (ob0, ob1, ob2, ob3, ob4, ob5)
    wid = lax.axis_index("s") * 2 + lax.axis_index("c")
    lane = lax.iota(jnp.int32, 16)
    zeros16 = jnp.zeros((16,), jnp.int32)

    pltpu.sync_copy(keys_hbm.at[pl.ds(wid * WPT, WPT)], keys_v)

    for g in range(CCAP // 16):
        ckey[pl.ds(g * 16, 16)] = zeros16
    def _compact(i, cnt):
        k = keys_v[pl.ds(i * 16, 16)]
        m = k > 0
        c = jnp.sum(m.astype(jnp.int32))
        @pl.when((c > 0) & (cnt <= CCAP - 16))
        def _():
            plsc.store_compressed(ckey.at[pl.ds(cnt, 16)], k, mask=m)
            plsc.store_compressed(cpos.at[pl.ds(cnt, 16)], i * 16 + lane, mask=m)
        return cnt + c
    npos = lax.fori_loop(0, VPT, _compact, jnp.int32(0))

    def _zero_hist():
        for g in range(512):
            hist[pl.ds(g * 16, 16)] = zeros16

    def _hist_pass(bin_fn, mask_fn):
        def _scan(i, _):
            k = ckey[pl.ds(i * 16, 16)]
            m = (k > 0) & mask_fn(k)
            b = bin_fn(k)
            idx = lane * 512 + b
            plsc.addupdate_scatter(hist, [idx], jnp.ones((16,), jnp.int32),
                                   mask=m)
            return 0
        lax.fori_loop(0, CCAP // 16, _scan, 0)

    def _suffix_pivot(target):
        def _grp(t, carry):
            found, piv, above, acc = carry
            g = 31 - t
            v = hist[pl.ds(g * 16, 16)]
            for r in range(1, 16):
                v = v + hist[pl.ds(r * 512 + g * 16, 16)]
            rsuf = lax.rev(plsc.cumsum(lax.rev(v, (0,))), (0,)) + acc
            m = rsuf >= target
            any_hit = jnp.sum(m.astype(jnp.int32)) > 0
            j = jnp.max(jnp.where(m, lane, -1))
            hit_cnt = jnp.sum(jnp.where(lane == j, v, 0))
            hit_suf = jnp.sum(jnp.where(lane == j, rsuf, 0))
            take = any_hit & jnp.logical_not(found)
            piv = jnp.where(take, g * 16 + j, piv)
            above = jnp.where(take, hit_suf - hit_cnt, above)
            found = found | any_hit
            acc = acc + jnp.sum(v)
            return found, piv, above, acc
        init = (jnp.bool_(False), jnp.int32(0), jnp.int32(0), jnp.int32(0))
        found, piv, above, _ = lax.fori_loop(0, 32, _grp, init)
        return found, piv, above

    _zero_hist()
    _hist_pass(lambda k: (k - KBASE) >> 17, lambda k: k == k)
    f1, b1, above1 = _suffix_pivot(TOPT)
    _zero_hist()
    _hist_pass(lambda k: ((k - KBASE) >> 8) & 0x1FF,
               lambda k: ((k - KBASE) >> 17) == b1)
    f2, b2, _ = _suffix_pivot(TOPT - above1)
    tkey = jnp.where(f1 & f2, KBASE + (b1 << 17) + (b2 << 8), jnp.int32(1))

    for g in range(SELCAP // 16):
        selkey[pl.ds(g * 16, 16)] = zeros16
        selpos[pl.ds(g * 16, 16)] = zeros16
    def _select(i, cnt):
        k = ckey[pl.ds(i * 16, 16)]
        m = k >= tkey
        c = jnp.sum(m.astype(jnp.int32))
        @pl.when((c > 0) & (cnt <= SELCAP - 16))
        def _():
            plsc.store_compressed(selkey.at[pl.ds(cnt, 16)], k, mask=m)
            p = cpos[pl.ds(i * 16, 16)]
            plsc.store_compressed(selpos.at[pl.ds(cnt, 16)], p, mask=m)
        return cnt + c
    lax.fori_loop(0, CCAP // 16, _select, jnp.int32(0))

    for g in range(SELCAP // 16):
        pos = selpos[pl.ds(g * 16, 16)]
        gpos = wid * WPT + pos
        n = gpos // KW
        cc = gpos % KW
        r, o = divmod(g * 16, 96)
        gidx[r][pl.ds(o, 16)] = (n * (C * 4) + 4 * cc) >> 4
        pidx[r][pl.ds(o, 16)] = n >> 2
    for r in range(3):
        pltpu.sync_copy(breg_hbm.at[gidx[r]], rel.at[pl.ds(r * 96, 96)])
        pltpu.sync_copy(prop_hbm.at[pidx[r]], prp.at[pl.ds(r * 96, 96)])

    for g in range(SELCAP // 16):
        rows = g * 16 + lane
        pos = selpos[pl.ds(g * 16, 16)]
        gpos = wid * WPT + pos
        n = gpos // KW
        cc = gpos % KW
        rr = (n * (C * 4) + 4 * cc) & 15
        rp = (n & 3) * 4
        r0 = plsc.load_gather(rel, [rows, rr])
        r1 = plsc.load_gather(rel, [rows, rr + 1])
        r2 = plsc.load_gather(rel, [rows, rr + 2])
        r3 = plsc.load_gather(rel, [rows, rr + 3])
        p0 = plsc.load_gather(prp, [rows, rp])
        p1 = plsc.load_gather(prp, [rows, rp + 1])
        p2 = plsc.load_gather(prp, [rows, rp + 2])
        p3 = plsc.load_gather(prp, [rows, rp + 3])
        w = p2 - p0
        h = p3 - p1
        cx = p0 + 0.5 * w
        cy = p1 + 0.5 * h
        dx = r0 / 10.0
        dy = r1 / 10.0
        dw = jnp.minimum(r2 / 5.0, BBOX_XFORM_CLIP)
        dh = jnp.minimum(r3 / 5.0, BBOX_XFORM_CLIP)
        pcx = dx * w + cx
        pcy = dy * h + cy
        pw = jnp.exp(dw) * w
        ph = jnp.exp(dh) * h
        x1 = jnp.minimum(jnp.maximum(pcx - 0.5 * pw, 0.0), IMG_W)
        x2 = jnp.minimum(jnp.maximum(pcx + 0.5 * pw, 0.0), IMG_W)
        y1 = jnp.minimum(jnp.maximum(pcy - 0.5 * ph, 0.0), IMG_H)
        y2 = jnp.minimum(jnp.maximum(pcy + 0.5 * ph, 0.0), IMG_H)
        k = selkey[pl.ds(g * 16, 16)]
        sc = plsc.bitcast(k, jnp.float32)
        pos = selpos[pl.ds(g * 16, 16)]
        lbf = ((wid * WPT + pos) % KW).astype(jnp.float32)
        doff = lbf * OFF
        sl = pl.ds(g * 16, 16)
        ob[0][sl] = x1 + doff
        ob[1][sl] = y1 + doff
        ob[2][sl] = x2 + doff
        ob[3][sl] = y2 + doff
        ob[4][sl] = sc
        ob[5][sl] = lbf

    dst = pl.ds(wid * SELCAP, SELCAP)
    pltpu.sync_copy(ob[0], ox1_hbm.at[dst])
    pltpu.sync_copy(ob[1], oy1_hbm.at[dst])
    pltpu.sync_copy(ob[2], ox2_hbm.at[dst])
    pltpu.sync_copy(ob[3], oy2_hbm.at[dst])
    pltpu.sync_copy(ob[4], osc_hbm.at[dst])
    pltpu.sync_copy(ob[5], olb_hbm.at[dst])


def _sc_select(keys_flat, breg2, proposals):
    f32 = jnp.float32
    mesh = plsc.VectorSubcoreMesh(core_axis_name="c", subcore_axis_name="s")
    fn = pl.kernel(
        _sc_body,
        mesh=mesh,
        compiler_params=pltpu.CompilerParams(needs_layout_passes=False,
                                             use_tc_tiling_on_sc=False),
        out_type=[jax.ShapeDtypeStruct((NCAND,), f32)] * 6,
        scratch_types=[
            pltpu.VMEM((WPT,), jnp.int32),
            pltpu.VMEM((CCAP,), jnp.int32),
            pltpu.VMEM((CCAP,), jnp.int32),
            pltpu.VMEM((16 * 512,), jnp.int32),
            pltpu.VMEM((SELCAP,), jnp.int32),
            pltpu.VMEM((SELCAP,), jnp.int32),
            pltpu.VMEM((96,), jnp.int32),
            pltpu.VMEM((96,), jnp.int32),
            pltpu.VMEM((96,), jnp.int32),
            pltpu.VMEM((96,), jnp.int32),
            pltpu.VMEM((96,), jnp.int32),
            pltpu.VMEM((96,), jnp.int32),
            pltpu.VMEM((SELCAP, 16), f32),
            pltpu.VMEM((SELCAP, 16), f32),
            pltpu.VMEM((SELCAP,), f32),
            pltpu.VMEM((SELCAP,), f32),
            pltpu.VMEM((SELCAP,), f32),
            pltpu.VMEM((SELCAP,), f32),
            pltpu.VMEM((SELCAP,), f32),
            pltpu.VMEM((SELCAP,), f32),
            pltpu.SemaphoreType.DMA,
        ],
    )
    return fn(keys_flat, breg2, proposals)


def _nms_body(ox1_ref, oy1_ref, ox2_ref, oy2_ref, sc_ref, lb_ref,
              *out_refs):
    ox1 = ox1_ref[...]
    oy1 = oy1_ref[...]
    ox2 = ox2_ref[...]
    oy2 = oy2_ref[...]
    lb = lb_ref[...]
    scores0 = sc_ref[...]
    areas = (ox2 - ox1) * (oy2 - oy1)
    fidx = lax.broadcasted_iota(jnp.int32, scores0.shape, 0) * 128 + \
        lax.broadcasted_iota(jnp.int32, scores0.shape, 1)
    oidx = lax.broadcasted_iota(jnp.int32, (8, 128), 0) * 128 + \
        lax.broadcasted_iota(jnp.int32, (8, 128), 1)
    zb = jnp.zeros((8, 128), jnp.float32)

    def step(i, carry):
        sw, b0, b1, b2, b3, b4, b5 = carry
        m = jnp.max(sw)
        eq = sw == m
        widx = jnp.min(jnp.where(eq, fidx, jnp.int32(2**30)))
        sel = fidx == widx
        wx1 = jnp.sum(jnp.where(sel, ox1, 0.0))
        wy1 = jnp.sum(jnp.where(sel, oy1, 0.0))
        wx2 = jnp.sum(jnp.where(sel, ox2, 0.0))
        wy2 = jnp.sum(jnp.where(sel, oy2, 0.0))
        wlb = jnp.sum(jnp.where(sel, lb, 0.0))
        warea = (wx2 - wx1) * (wy2 - wy1)
        ltx = jnp.maximum(wx1, ox1)
        lty = jnp.maximum(wy1, oy1)
        rbx = jnp.minimum(wx2, ox2)
        rby = jnp.minimum(wy2, oy2)
        iw = jnp.maximum(rbx - ltx, 0.0)
        ih = jnp.maximum(rby - lty, 0.0)
        inter = iw * ih
        iou = inter / (warea + areas - inter + 1e-9)
        sw = jnp.where(iou > NMS_THRESH, NEG_INF, sw)
        sw = jnp.where(sel, NEG_INF, sw)
        valid = jnp.where(m > 0.0, 1.0, 0.0)
        doff = wlb * OFF
        here = oidx == i
        b0 = jnp.where(here, (wx1 - doff) * valid, b0)
        b1 = jnp.where(here, (wy1 - doff) * valid, b1)
        b2 = jnp.where(here, (wx2 - doff) * valid, b2)
        b3 = jnp.where(here, (wy2 - doff) * valid, b3)
        b4 = jnp.where(here, m * valid, b4)
        b5 = jnp.where(here, wlb * valid, b5)
        return sw, b0, b1, b2, b3, b4, b5

    carry = (scores0, zb, zb, zb, zb, zb, zb)
    carry = lax.fori_loop(0, DET_PER_IMG, step, carry)
    for r, v in zip(out_refs, carry[1:]):
        r[...] = v


def _nms(ox1, oy1, ox2, oy2, scores, labels_f):
    K = scores.shape[0]
    R = K // 128
    args = [a.reshape(R, 128) for a in (ox1, oy1, ox2, oy2, scores, labels_f)]
    outs = pl.pallas_call(
        _nms_body,
        in_specs=[pl.BlockSpec((R, 128), lambda: (0, 0))] * 6,
        out_specs=[pl.BlockSpec((8, 128), lambda: (0, 0))] * 6,
        out_shape=[jax.ShapeDtypeStruct((8, 128), jnp.float32)] * 6,
    )(*args)
    return [o.reshape(-1)[:DET_PER_IMG] for o in outs]


def kernel(class_logits, box_regression, proposals):
    keys = _score_keys(class_logits)
    breg16 = box_regression.reshape(N * C * 4 // 16, 16)
    prop16 = proposals.reshape(N * 4 // 16, 16)
    o = _sc_select(keys.reshape(-1), breg16, prop16)
    d = _nms(o[0], o[1], o[2], o[3], o[4], o[5])
    dets = jnp.stack([d[0], d[1], d[2], d[3], d[4]], axis=1)
    det_labels = d[5].astype(jnp.int32)
    return dets, det_labels
